# baseline (speedup 1.0000x reference)
import functools

import jax
import jax.numpy as jnp
from jax import lax
from jax.experimental import pallas as pl
from jax.experimental.pallas import tpu as pltpu
from jax.experimental.pallas import tpu_sc as plsc

_B = 16384
_D = 64
_NEG = 20

_NC = 2
_NS = 16
_NW = _NC * _NS
_L = 16
_G = _D // _L

_BPW = _B // _NW
_CH = 64
_NCH = _BPW // _CH
_IDX_CH = 128


def _sc_body(ii_hbm, pi_hbm, ni_hbm, ev_hbm, eu_hbm,
             pos_out, neg_out,
             idx_in, idx_pos, idx_neg, v_buf, p_buf, n_buf,
             pos_buf, neg_buf, sem):
    w = lax.axis_index("s") * _NC + lax.axis_index("c")
    base = w * _BPW

    pltpu.sync_copy(ii_hbm.at[pl.ds(base, _BPW)], idx_in)
    pltpu.sync_copy(pi_hbm.at[pl.ds(base, _BPW)], idx_pos)
    pltpu.sync_copy(ni_hbm.at[pl.ds(base * _NEG, _BPW * _NEG)], idx_neg)

    def chunk_body(c, _):
        cb = pl.multiple_of(c * _CH, _CH)
        cps = [
            pltpu.async_copy(ev_hbm.at[idx_in.at[pl.ds(cb, _CH)]], v_buf, sem),
            pltpu.async_copy(eu_hbm.at[idx_pos.at[pl.ds(cb, _CH)]], p_buf, sem),
        ]
        nbase = pl.multiple_of(c * (_CH * _NEG), _CH * _NEG)
        for k in range(_CH * _NEG // _IDX_CH):
            cps.append(pltpu.async_copy(
                eu_hbm.at[idx_neg.at[pl.ds(nbase + k * _IDX_CH, _IDX_CH)]],
                n_buf.at[pl.ds(k * _IDX_CH, _IDX_CH)], sem))
        for cp in cps:
            cp.wait()

        def b_body(b, _):
            nb = b * _NEG
            ob = cb + b
            pos_acc = None
            neg_acc = None
            for g in range(_G):
                sl = pl.ds(g * _L, _L)
                vv = v_buf[b, sl]
                t = n_buf[nb, sl]
                for j in range(1, _NEG):
                    t = t + n_buf[nb + j, sl]
                pa = vv * p_buf[b, sl]
                na = vv * t
                pos_acc = pa if g == 0 else pos_acc + pa
                neg_acc = na if g == 0 else neg_acc + na
            pos_buf[ob, :] = pos_acc
            neg_buf[ob, :] = neg_acc
            return 0

        lax.fori_loop(0, _CH, b_body, 0)
        return 0

    lax.fori_loop(0, _NCH, chunk_body, 0)
    pltpu.sync_copy(pos_buf, pos_out.at[pl.ds(base, _BPW)])
    pltpu.sync_copy(neg_buf, neg_out.at[pl.ds(base, _BPW)])


@functools.cache
def _sc_scores():
  return pl.kernel(
    _sc_body,
    out_type=(
        jax.ShapeDtypeStruct((_B, _L), jnp.float32),
        jax.ShapeDtypeStruct((_B, _L), jnp.float32),
    ),
    mesh=plsc.VectorSubcoreMesh(core_axis_name="c", subcore_axis_name="s",
                                num_cores=_NC, num_subcores=_NS),
    compiler_params=pltpu.CompilerParams(use_tc_tiling_on_sc=False),
    scratch_types=[
        pltpu.VMEM((_BPW,), jnp.int32),
        pltpu.VMEM((_BPW,), jnp.int32),
        pltpu.VMEM((_BPW * _NEG,), jnp.int32),
        pltpu.VMEM((_CH, _D), jnp.float32),
        pltpu.VMEM((_CH, _D), jnp.float32),
        pltpu.VMEM((_CH * _NEG, _D), jnp.float32),
        pltpu.VMEM((_BPW, _L), jnp.float32),
        pltpu.VMEM((_BPW, _L), jnp.float32),
        pltpu.SemaphoreType.DMA,
    ],
  )


def _log_sigmoid(x):
    return jnp.minimum(x, 0.0) - jnp.log1p(jnp.exp(-jnp.abs(x)))


def _loss_body(pos_ref, neg_ref, out_ref):
    pos = jnp.sum(pos_ref[...], axis=1, keepdims=True)
    neg = -jnp.sum(neg_ref[...], axis=1, keepdims=True)
    loss = _log_sigmoid(pos) + _log_sigmoid(neg)
    out_ref[...] = -jnp.sum(loss, axis=(0, 1), keepdims=True) / _B


_tc_loss = pl.pallas_call(
    _loss_body,
    out_shape=jax.ShapeDtypeStruct((1, 1), jnp.float32),
)


def kernel(input_items, pos_items, neg_items, embedding_v, embedding_u):
    ii = input_items.reshape(_B)
    pi = pos_items.reshape(_B)
    ni = neg_items.reshape(_B * _NEG)
    pos_part, neg_part = _sc_scores()(ii, pi, ni, embedding_v, embedding_u)
    return _tc_loss(pos_part, neg_part).reshape(())


# speedup vs baseline: 1.5019x; 1.5019x over previous
import functools

import jax
import jax.numpy as jnp
from jax import lax
from jax.experimental import pallas as pl
from jax.experimental.pallas import tpu as pltpu
from jax.experimental.pallas import tpu_sc as plsc
from jax.experimental.layout import Format, Layout, with_layout_constraint

_B = 16384
_D = 64
_NEG = 20
_ITEMS = 1000000

_NC = 2
_NS = 16
_NW = _NC * _NS
_L = 16
_G = _D // _L

_BPW = _B // _NW
_CH = 64
_NCH = _BPW // _CH
_IDX_CH = 128


def _sc_body(ii_hbm, pi_hbm, ni_hbm, ev_hbm, eu_hbm,
             pos_out, neg_out,
             idx_in, idx_pos, idx_neg, v_buf, p_buf, n_buf,
             pos_buf, neg_buf, sem):
    w = lax.axis_index("s") * _NC + lax.axis_index("c")
    base = w * _BPW

    pltpu.sync_copy(ii_hbm.at[pl.ds(base, _BPW)], idx_in)
    pltpu.sync_copy(pi_hbm.at[pl.ds(base, _BPW)], idx_pos)
    pltpu.sync_copy(ni_hbm.at[pl.ds(base * _NEG, _BPW * _NEG)], idx_neg)

    def chunk_body(c, _):
        cb = pl.multiple_of(c * _CH, _CH)
        cps = [
            pltpu.async_copy(ev_hbm.at[idx_in.at[pl.ds(cb, _CH)]], v_buf, sem),
            pltpu.async_copy(eu_hbm.at[idx_pos.at[pl.ds(cb, _CH)]], p_buf, sem),
        ]
        nbase = pl.multiple_of(c * (_CH * _NEG), _CH * _NEG)
        for k in range(_CH * _NEG // _IDX_CH):
            cps.append(pltpu.async_copy(
                eu_hbm.at[idx_neg.at[pl.ds(nbase + k * _IDX_CH, _IDX_CH)]],
                n_buf.at[pl.ds(k * _IDX_CH, _IDX_CH)], sem))
        for cp in cps:
            cp.wait()

        def b_body(b, _):
            nb = b * _NEG
            ob = cb + b
            pos_acc = None
            neg_acc = None
            for g in range(_G):
                sl = pl.ds(g * _L, _L)
                vv = v_buf[b, sl]
                t = n_buf[nb, sl]
                for j in range(1, _NEG):
                    t = t + n_buf[nb + j, sl]
                pa = vv * p_buf[b, sl]
                na = vv * t
                pos_acc = pa if g == 0 else pos_acc + pa
                neg_acc = na if g == 0 else neg_acc + na
            pos_buf[ob, :] = pos_acc
            neg_buf[ob, :] = neg_acc
            return 0

        lax.fori_loop(0, _CH, b_body, 0)
        return 0

    lax.fori_loop(0, _NCH, chunk_body, 0)
    pltpu.sync_copy(pos_buf, pos_out.at[pl.ds(base, _BPW)])
    pltpu.sync_copy(neg_buf, neg_out.at[pl.ds(base, _BPW)])


@functools.cache
def _sc_scores():
  return pl.kernel(
    _sc_body,
    out_type=(
        jax.ShapeDtypeStruct((_B, _L), jnp.float32),
        jax.ShapeDtypeStruct((_B, _L), jnp.float32),
    ),
    mesh=plsc.VectorSubcoreMesh(core_axis_name="c", subcore_axis_name="s",
                                num_cores=_NC, num_subcores=_NS),
    compiler_params=pltpu.CompilerParams(use_tc_tiling_on_sc=False),
    scratch_types=[
        pltpu.VMEM((_BPW,), jnp.int32),
        pltpu.VMEM((_BPW,), jnp.int32),
        pltpu.VMEM((_BPW * _NEG,), jnp.int32),
        pltpu.VMEM((_CH, _D), jnp.float32),
        pltpu.VMEM((_CH, _D), jnp.float32),
        pltpu.VMEM((_CH * _NEG, _D), jnp.float32),
        pltpu.VMEM((_BPW, _L), jnp.float32),
        pltpu.VMEM((_BPW, _L), jnp.float32),
        pltpu.SemaphoreType.DMA,
    ],
  )


def _log_sigmoid(x):
    return jnp.minimum(x, 0.0) - jnp.log1p(jnp.exp(-jnp.abs(x)))


def _loss_body(pos_ref, neg_ref, out_ref):
    pos = jnp.sum(pos_ref[...], axis=1, keepdims=True)
    neg = -jnp.sum(neg_ref[...], axis=1, keepdims=True)
    loss = _log_sigmoid(pos) + _log_sigmoid(neg)
    out_ref[...] = -jnp.sum(loss, axis=(0, 1), keepdims=True) / _B


_tc_loss = pl.pallas_call(
    _loss_body,
    out_shape=jax.ShapeDtypeStruct((1, 1), jnp.float32),
)


def kernel(input_items, pos_items, neg_items, embedding_v, embedding_u):
    ii = input_items.reshape(_B)
    pi = pos_items.reshape(_B)
    ni = neg_items.reshape(_B * _NEG)
    fmt = Layout(major_to_minor=(0, 1), tiling=((8,),))
    ev = with_layout_constraint(embedding_v, fmt)
    eu = with_layout_constraint(embedding_u, fmt)
    pos_part, neg_part = _sc_scores()(ii, pi, ni, ev, eu)
    return _tc_loss(pos_part, neg_part).reshape(())
